# baseline (speedup 1.0000x reference)
import functools

import jax
import jax.numpy as jnp
from jax.experimental import pallas as pl
from jax.experimental.pallas import tpu as pltpu

EPS = 1e-5


def _rb_kernel(x_ref, w1_ref, w2_ref, o_ref, x3_ref, y_ref, h_ref, *, H, W):
    C = x_ref.shape[0]
    HW = H * W
    xflat = x_ref[...].reshape(C, HW)
    inv_n = 1.0 / HW

    lane = jax.lax.broadcasted_iota(jnp.int32, (1, HW), 1)
    col = lane % W
    mask_l = col == 0
    mask_r = col == (W - 1)
    mask_top = lane < W
    mask_bot = lane >= (H - 1) * W

    def build_x3(src):
        rp = pltpu.roll(src, W, 1)
        rm = pltpu.roll(src, HW - W, 1)
        x3_ref[0:C, :] = jnp.where(mask_top, rm, rp)
        x3_ref[C:2 * C, :] = src
        x3_ref[2 * C:3 * C, :] = jnp.where(mask_bot, rp, rm)

    def conv(w_ref):
        y_ref[...] = jnp.dot(w_ref[...], x3_ref[...],
                             preferred_element_type=jnp.float32
                             ).astype(y_ref.dtype)
        c0 = y_ref[0:C, :]
        c1 = y_ref[C:2 * C, :]
        c2 = y_ref[2 * C:3 * C, :]
        p0 = pltpu.roll(c0, 1, 1)
        m0 = pltpu.roll(c0, HW - 1, 1)
        p2 = pltpu.roll(c2, 1, 1)
        m2 = pltpu.roll(c2, HW - 1, 1)
        y = (c1 + jnp.where(mask_l, m0, p0)
             + jnp.where(mask_r, p2, m2))
        h_ref[...] = y
        yf = y.astype(jnp.float32)
        s = jnp.sum(yf, axis=1, keepdims=True)
        ss = jnp.sum(yf * yf, axis=1, keepdims=True)
        mean = s * inv_n
        var = jnp.maximum(ss * inv_n - mean * mean, 0.0)
        return mean, jax.lax.rsqrt(var + EPS)

    build_x3(xflat.astype(x3_ref.dtype))
    mean1, scale1 = conv(w1_ref)

    h1 = jnp.maximum((h_ref[...] - mean1.astype(h_ref.dtype))
                     * scale1.astype(h_ref.dtype), 0.0)
    build_x3(h1)

    mean2, scale2 = conv(w2_ref)

    h2 = (h_ref[...].astype(jnp.float32) - mean2) * scale2
    o_ref[...] = ((xflat.astype(jnp.float32) + h2)
                  .astype(o_ref.dtype).reshape(o_ref.shape))


def kernel(x, w1, b1, w2, b2):
    del b1, b2
    N, C, H, W = x.shape
    if H < 2 or W < 2:
        raise ValueError("reflect padding of 1 requires H >= 2 and W >= 2")

    if (2 * W) % 128 == 0 and H % 2 == 0:
        Hb, Wb = H // 2, 2 * W
    else:
        Hb, Wb = H, W
    xf = x.reshape(N, C, Hb, Wb)

    def prep(w):
        t = jnp.transpose(w, (3, 0, 2, 1))
        return t.reshape(3 * C, 3 * C).astype(jnp.bfloat16)

    out = pl.pallas_call(
        functools.partial(_rb_kernel, H=H, W=W),
        out_shape=jax.ShapeDtypeStruct((N, C, Hb, Wb), x.dtype),
        grid=(N,),
        in_specs=[
            pl.BlockSpec((None, C, Hb, Wb), lambda n: (n, 0, 0, 0)),
            pl.BlockSpec((3 * C, 3 * C), lambda n: (0, 0)),
            pl.BlockSpec((3 * C, 3 * C), lambda n: (0, 0)),
        ],
        out_specs=pl.BlockSpec((None, C, Hb, Wb), lambda n: (n, 0, 0, 0)),
        scratch_shapes=[
            pltpu.VMEM((3 * C, H * W), jnp.bfloat16),
            pltpu.VMEM((3 * C, H * W), jnp.bfloat16),
            pltpu.VMEM((C, H * W), jnp.bfloat16),
        ],
        compiler_params=pltpu.CompilerParams(
            dimension_semantics=("parallel",),
            vmem_limit_bytes=48 * 1024 * 1024),
    )(xf, prep(w1), prep(w2))

    return out.reshape(N, C, H, W)


# speedup vs baseline: 1.1062x; 1.1062x over previous
import functools

import jax
import jax.numpy as jnp
from jax.experimental import pallas as pl
from jax.experimental.pallas import tpu as pltpu

EPS = 1e-5


def _rb_kernel(x_ref, w1_ref, w2_ref, o_ref, x3_ref, y_ref, h_ref, *, H, W):
    C = x_ref.shape[0]
    HW = H * W
    inv_n = 1.0 / HW

    lane = jax.lax.broadcasted_iota(jnp.int32, (1, HW), 1)
    col = lane % W
    mask_l = col == 0
    mask_r = col == (W - 1)
    mask_top = lane < W
    mask_bot = lane >= (H - 1) * W

    def build_x3(src):
        rp = pltpu.roll(src, W, 1)
        rm = pltpu.roll(src, HW - W, 1)
        x3_ref[0:C, :] = jnp.where(mask_top, rm, rp)
        x3_ref[C:2 * C, :] = src
        x3_ref[2 * C:3 * C, :] = jnp.where(mask_bot, rp, rm)

    def conv(w_ref):
        y_ref[...] = jnp.dot(w_ref[...], x3_ref[...],
                             preferred_element_type=jnp.float32
                             ).astype(y_ref.dtype)
        c0 = y_ref[0:C, :]
        c1 = y_ref[C:2 * C, :]
        c2 = y_ref[2 * C:3 * C, :]
        p0 = pltpu.roll(c0, 1, 1)
        m0 = pltpu.roll(c0, HW - 1, 1)
        p2 = pltpu.roll(c2, 1, 1)
        m2 = pltpu.roll(c2, HW - 1, 1)
        y = (c1.astype(jnp.float32)
             + jnp.where(mask_l, m0, p0).astype(jnp.float32)
             + jnp.where(mask_r, p2, m2).astype(jnp.float32))
        h_ref[...] = y.astype(h_ref.dtype)
        s = jnp.sum(y, axis=1, keepdims=True)
        ss = jnp.sum(y * y, axis=1, keepdims=True)
        mean = s * inv_n
        var = jnp.maximum(ss * inv_n - mean * mean, 0.0)
        return mean, jax.lax.rsqrt(var + EPS)

    build_x3(x_ref[...].astype(x3_ref.dtype))
    mean1, scale1 = conv(w1_ref)

    h1 = jnp.maximum((h_ref[...].astype(jnp.float32) - mean1) * scale1, 0.0)
    build_x3(h1.astype(x3_ref.dtype))

    mean2, scale2 = conv(w2_ref)

    h2 = (h_ref[...].astype(jnp.float32) - mean2) * scale2
    o_ref[...] = (x_ref[...].astype(jnp.float32) + h2).astype(o_ref.dtype)


def kernel(x, w1, b1, w2, b2):
    del b1, b2
    N, C, H, W = x.shape
    if H < 2 or W < 2:
        raise ValueError("reflect padding of 1 requires H >= 2 and W >= 2")

    xf = x.reshape(N, C, H * W)

    def prep(w):
        t = jnp.transpose(w, (3, 0, 2, 1))
        return t.reshape(3 * C, 3 * C).astype(jnp.bfloat16)

    out = pl.pallas_call(
        functools.partial(_rb_kernel, H=H, W=W),
        out_shape=jax.ShapeDtypeStruct((N, C, H * W), x.dtype),
        grid=(N,),
        in_specs=[
            pl.BlockSpec((None, C, H * W), lambda n: (n, 0, 0)),
            pl.BlockSpec((3 * C, 3 * C), lambda n: (0, 0)),
            pl.BlockSpec((3 * C, 3 * C), lambda n: (0, 0)),
        ],
        out_specs=pl.BlockSpec((None, C, H * W), lambda n: (n, 0, 0)),
        scratch_shapes=[
            pltpu.VMEM((3 * C, H * W), jnp.bfloat16),
            pltpu.VMEM((3 * C, H * W), jnp.bfloat16),
            pltpu.VMEM((C, H * W), jnp.bfloat16),
        ],
        compiler_params=pltpu.CompilerParams(
            dimension_semantics=("parallel",),
            vmem_limit_bytes=48 * 1024 * 1024),
    )(xf, prep(w1), prep(w2))

    return out.reshape(N, C, H, W)


# speedup vs baseline: 1.1337x; 1.0249x over previous
import functools

import jax
import jax.numpy as jnp
from jax.experimental import pallas as pl
from jax.experimental.pallas import tpu as pltpu

EPS = 1e-5


def _rb_kernel(x_ref, w1_ref, w2_ref, o_ref, x3_ref, y_ref, h_ref, *, H, W):
    B, C = x_ref.shape[0], x_ref.shape[1]
    HW = H * W
    inv_n = 1.0 / HW

    lane = jax.lax.broadcasted_iota(jnp.int32, (1, HW), 1)
    col = lane % W
    mask_l = col == 0
    mask_r = col == (W - 1)
    mask_top = lane < W
    mask_bot = lane >= (H - 1) * W

    def build_x3(src):
        rp = pltpu.roll(src, W, 1)
        rm = pltpu.roll(src, HW - W, 1)
        x3_ref[0:C, :] = jnp.where(mask_top, rm, rp)
        x3_ref[C:2 * C, :] = src
        x3_ref[2 * C:3 * C, :] = jnp.where(mask_bot, rp, rm)

    def conv(w_ref):
        y_ref[...] = jnp.dot(w_ref[...], x3_ref[...],
                             preferred_element_type=jnp.float32
                             ).astype(y_ref.dtype)
        c0 = y_ref[0:C, :]
        c1 = y_ref[C:2 * C, :]
        c2 = y_ref[2 * C:3 * C, :]
        p0 = pltpu.roll(c0, 1, 1)
        m0 = pltpu.roll(c0, HW - 1, 1)
        p2 = pltpu.roll(c2, 1, 1)
        m2 = pltpu.roll(c2, HW - 1, 1)
        y = (c1.astype(jnp.float32)
             + jnp.where(mask_l, m0, p0).astype(jnp.float32)
             + jnp.where(mask_r, p2, m2).astype(jnp.float32))
        h_ref[...] = y.astype(h_ref.dtype)
        s = jnp.sum(y, axis=1, keepdims=True)
        ss = jnp.sum(y * y, axis=1, keepdims=True)
        mean = s * inv_n
        var = jnp.maximum(ss * inv_n - mean * mean, 0.0)
        return mean, jax.lax.rsqrt(var + EPS)

    for b in range(B):
        build_x3(x_ref[b].astype(x3_ref.dtype))
        mean1, scale1 = conv(w1_ref)

        h1 = jnp.maximum((h_ref[...].astype(jnp.float32) - mean1) * scale1,
                         0.0)
        build_x3(h1.astype(x3_ref.dtype))

        mean2, scale2 = conv(w2_ref)

        h2 = (h_ref[...].astype(jnp.float32) - mean2) * scale2
        o_ref[b] = (x_ref[b].astype(jnp.float32) + h2).astype(o_ref.dtype)


def kernel(x, w1, b1, w2, b2):
    del b1, b2
    N, C, H, W = x.shape
    if H < 2 or W < 2:
        raise ValueError("reflect padding of 1 requires H >= 2 and W >= 2")

    xf = x.reshape(N, C, H * W)

    def prep(w):
        t = jnp.transpose(w, (3, 0, 2, 1))
        return t.reshape(3 * C, 3 * C).astype(jnp.bfloat16)

    B = 2 if N % 2 == 0 else 1
    out = pl.pallas_call(
        functools.partial(_rb_kernel, H=H, W=W),
        out_shape=jax.ShapeDtypeStruct((N, C, H * W), x.dtype),
        grid=(N // B,),
        in_specs=[
            pl.BlockSpec((B, C, H * W), lambda n: (n, 0, 0)),
            pl.BlockSpec((3 * C, 3 * C), lambda n: (0, 0)),
            pl.BlockSpec((3 * C, 3 * C), lambda n: (0, 0)),
        ],
        out_specs=pl.BlockSpec((B, C, H * W), lambda n: (n, 0, 0)),
        scratch_shapes=[
            pltpu.VMEM((3 * C, H * W), jnp.bfloat16),
            pltpu.VMEM((3 * C, H * W), jnp.bfloat16),
            pltpu.VMEM((C, H * W), jnp.bfloat16),
        ],
        compiler_params=pltpu.CompilerParams(
            dimension_semantics=("parallel",),
            vmem_limit_bytes=48 * 1024 * 1024),
    )(xf, prep(w1), prep(w2))

    return out.reshape(N, C, H, W)
